# baseline (speedup 1.0000x reference)
import functools

import jax
import jax.numpy as jnp
from jax import lax
from jax.experimental import pallas as pl
from jax.experimental.pallas import tpu as pltpu
from jax.experimental.pallas import tpu_sc as plsc

B = 16384
F = 64
N_ROWS = 1000000

_info = plsc.get_sparse_core_info()
NC = _info.num_cores
NS = _info.num_subcores
L = _info.num_lanes
NW = NC * NS
BPW = B // NW
CH = 128
NCH = BPW // CH

_mesh = plsc.VectorSubcoreMesh(core_axis_name="c", subcore_axis_name="s")


@functools.partial(
    pl.kernel,
    mesh=_mesh,
    compiler_params=pltpu.CompilerParams(needs_layout_passes=False),
    out_type=jax.ShapeDtypeStruct((B,), jnp.float32),
    scratch_types=[
        pltpu.VMEM((BPW,), jnp.int32),
        pltpu.VMEM((BPW,), jnp.int32),
        pltpu.VMEM((NCH, CH), jnp.int32),
        pltpu.VMEM((NCH, CH), jnp.int32),
        pltpu.VMEM((2, CH, 128), jnp.float32),
        pltpu.VMEM((2, CH, 128), jnp.float32),
        pltpu.VMEM((BPW,), jnp.float32),
        pltpu.SemaphoreType.DMA,
        pltpu.SemaphoreType.DMA,
    ],
)
def _mf_sc(user_hbm, item_hbm, uf2_hbm, if2_hbm, out_hbm,
           uh, ih, upr, ipr, ue, ie, outv, sem0, sem1):
    wid = lax.axis_index("s") * NC + lax.axis_index("c")

    pltpu.sync_copy(user_hbm.at[wid], uh)
    pltpu.sync_copy(item_hbm.at[wid], ih)

    def base_body(j, _):
        sl = pl.ds(j * L, L)
        k = j // (CH // L)
        p = j % (CH // L)
        csl = pl.ds(p * L, L)
        r = uh[sl]
        upr[k, csl] = r >> 1
        uh[sl] = (r & 1) << 6
        r = ih[sl]
        ipr[k, csl] = r >> 1
        ih[sl] = (r & 1) << 6
        return 0

    lax.fori_loop(0, BPW // L, base_body, 0)

    sems = (sem0, sem1)

    def fetch(k, buf):
        pltpu.async_copy(uf2_hbm.at[upr.at[k]], ue.at[buf], sems[buf])
        pltpu.async_copy(if2_hbm.at[ipr.at[k]], ie.at[buf], sems[buf])

    def drain(k, buf):
        pltpu.make_async_copy(uf2_hbm.at[upr.at[k]], ue.at[buf], sems[buf]).wait()
        pltpu.make_async_copy(if2_hbm.at[ipr.at[k]], ie.at[buf], sems[buf]).wait()

    lanes = lax.iota(jnp.int32, L)

    fetch(0, 0)
    for k in range(NCH):
        buf = k % 2
        drain(k, buf)
        if k + 1 < NCH:
            fetch(k + 1, (k + 1) % 2)

        def group_body(p, _, k=k, buf=buf):
            rows = p * L + lanes
            bsl = pl.ds(k * CH + p * L, L)
            hu = uh[bsl]
            hi = ih[bsl]
            acc = jnp.zeros((L,), jnp.float32)
            for j in range(F):
                jv = jnp.full((L,), j, jnp.int32)
                u16 = plsc.load_gather(ue.at[buf], [rows, hu + jv])
                v16 = plsc.load_gather(ie.at[buf], [rows, hi + jv])
                acc = acc + u16 * v16
            outv[bsl] = acc
            return 0

        lax.fori_loop(0, CH // L, group_body, 0)

    pltpu.sync_copy(outv, out_hbm.at[pl.ds(wid * BPW, BPW)])


def kernel(user, item, user_factors, item_factors):
    user_r = user.astype(jnp.int32).reshape(NW, BPW)
    item_r = item.astype(jnp.int32).reshape(NW, BPW)
    uf2 = user_factors.reshape(N_ROWS // 2, 2 * F)
    if2 = item_factors.reshape(N_ROWS // 2, 2 * F)
    return _mf_sc(user_r, item_r, uf2, if2)
